# Initial kernel scaffold; baseline (speedup 1.0000x reference)
import jax
import jax.numpy as jnp
from jax.experimental import pallas as pl


def kernel(x, edge_index, pos_edge_index, neg_edge_index, W1, b1, W2, b2):
    raise NotImplementedError("write your pallas kernel here")


# speedup vs baseline: 3.6459x; 3.6459x over previous
import functools

import jax
import jax.numpy as jnp
from jax import lax
from jax.experimental import pallas as pl
from jax.experimental.pallas import tpu as pltpu
from jax.experimental.pallas import tpu_sc as plsc

N = 50000
E = 800000
D = 64
NC, NS, L = 2, 16, 16
NW = NC * NS
HALF = N // NC
HALFP = HALF + 8
NP = 50176
B = 128
PE = 819200
ROWS = PE // B
RPT = ROWS // NW
RPS = ROWS // NS

_mesh = plsc.VectorSubcoreMesh(core_axis_name="c", subcore_axis_name="s")


def _zero16f():
    return jnp.zeros((L,), jnp.float32)


NPS = NP // NS


@functools.partial(
    pl.kernel,
    out_type=jax.ShapeDtypeStruct((NC * NP,), jnp.float32),
    mesh=_mesh,
    scratch_types=[
        pltpu.VMEM((NPS,), jnp.float32),
        pltpu.VMEM((B,), jnp.float32),
        pltpu.VMEM((B,), jnp.int32),
        pltpu.VMEM((B,), jnp.int32),
        pltpu.VMEM_SHARED((NP,), jnp.float32),
    ],
)
def _deg_kernel(dst_hbm, deg_out, stage_v, ones_v, didx_v, sidx_v, deg_sh):
    c = lax.axis_index("c")
    s = lax.axis_index("s")
    wid = s * NC + c

    one16 = jnp.ones((L,), jnp.float32)
    for k in range(B // L):
        ones_v[pl.ds(k * L, L)] = one16

    def _zero(i, _):
        stage_v[pl.ds(i * L, L)] = _zero16f()
        return 0

    lax.fori_loop(0, NPS // L, _zero, 0)
    pltpu.sync_copy(stage_v, deg_sh.at[pl.ds(s * NPS, NPS)])
    plsc.subcore_barrier()

    row0 = wid * RPT
    trash = jnp.full((L,), N, jnp.int32)

    def _batch(b, _):
        pltpu.sync_copy(dst_hbm.at[row0 + b], didx_v)
        for k in range(B // L):
            dv = didx_v[pl.ds(k * L, L)]
            sidx_v[pl.ds(k * L, L)] = jnp.where(dv >= 0, dv, trash)
        pltpu.sync_copy(ones_v, deg_sh.at[sidx_v], add=True)
        return 0

    lax.fori_loop(0, RPT, _batch, 0)
    plsc.subcore_barrier()

    pltpu.sync_copy(deg_sh.at[pl.ds(s * NPS, NPS)], stage_v)
    pltpu.sync_copy(stage_v, deg_out.at[pl.ds(c * NP + s * NPS, NPS)])


@functools.partial(
    pl.kernel,
    out_type=jax.ShapeDtypeStruct((N, D), jnp.float32),
    mesh=_mesh,
    compiler_params=pltpu.CompilerParams(use_tc_tiling_on_sc=False,
                                         needs_layout_passes=False),
    scratch_types=[
        pltpu.VMEM((B, D), jnp.float32),
        pltpu.VMEM((B,), jnp.int32),
        pltpu.VMEM((B,), jnp.int32),
        pltpu.VMEM((B,), jnp.int32),
        pltpu.VMEM_SHARED((HALFP, D), jnp.float32),
        pltpu.SemaphoreType.DMA,
    ],
)
def _conv_kernel(g_hbm, src_hbm, dst_hbm, acc_out, rows_v, gidx_v, didx_v,
                 sidx_v, acc_sh, sem):
    c = lax.axis_index("c")
    s = lax.axis_index("s")

    def _zrow(i, _):
        for k in range(D // L):
            rows_v[i, pl.ds(k * L, L)] = _zero16f()
        return 0

    lax.fori_loop(0, B, _zrow, 0)

    zr0 = s * (HALFP // NS)

    def _zacc(i, _):
        pltpu.sync_copy(rows_v, acc_sh.at[pl.ds(zr0 + i * B, B)])
        return 0

    lax.fori_loop(0, 12, _zacc, 0)
    pltpu.sync_copy(rows_v.at[pl.ds(0, 27)],
                    acc_sh.at[pl.ds(zr0 + 12 * B, 27)])
    plsc.subcore_barrier()

    lo = c * HALF
    trash = jnp.full((L,), HALF, jnp.int32)
    row0 = s * RPS

    def _batch(b, _):
        row = row0 + b
        pltpu.sync_copy(src_hbm.at[row], gidx_v)
        pltpu.sync_copy(dst_hbm.at[row], didx_v)
        pltpu.async_copy(g_hbm.at[gidx_v], rows_v, sem).wait()
        for k in range(B // L):
            dv = didx_v[pl.ds(k * L, L)]
            loc = dv - lo
            ok = (loc >= 0) & (loc < HALF)
            sidx_v[pl.ds(k * L, L)] = jnp.where(ok, loc, trash)
        pltpu.sync_copy(rows_v, acc_sh.at[sidx_v], add=True)
        return 0

    lax.fori_loop(0, RPS, _batch, 0)
    plsc.subcore_barrier()

    out0 = c * HALF
    cr0 = s * 1568

    @pl.when(s < NS - 1)
    def _():
        pltpu.sync_copy(acc_sh.at[pl.ds(cr0, 1568)],
                        acc_out.at[pl.ds(out0 + cr0, 1568)])

    @pl.when(s == NS - 1)
    def _():
        pltpu.sync_copy(acc_sh.at[pl.ds(cr0, 1480)],
                        acc_out.at[pl.ds(out0 + cr0, 1480)])


@functools.partial(
    pl.kernel,
    out_type=(jax.ShapeDtypeStruct((PE,), jnp.float32),
              jax.ShapeDtypeStruct((PE,), jnp.float32)),
    mesh=_mesh,
    compiler_params=pltpu.CompilerParams(use_tc_tiling_on_sc=False,
                                         needs_layout_passes=False),
    scratch_types=[
        pltpu.VMEM((B, D), jnp.float32),
        pltpu.VMEM((B, D), jnp.float32),
        pltpu.VMEM((B,), jnp.int32),
        pltpu.VMEM((B,), jnp.int32),
        pltpu.VMEM((B,), jnp.float32),
        pltpu.SemaphoreType.DMA,
        pltpu.SemaphoreType.DMA,
    ],
)
def _score_kernel(z_hbm, pa_hbm, pb_hbm, na_hbm, nb_hbm, pos_out, neg_out,
                  za_v, zb_v, aidx_v, bidx_v, scr_v, sema, semb):
    c = lax.axis_index("c")
    s = lax.axis_index("s")
    wid = s * NC + c
    row0 = wid * RPT
    lanes = lax.iota(jnp.int32, L)

    def _polarity(a_hbm, b_hbm, out_hbm):
        def _batch(b, _):
            row = row0 + b
            pltpu.sync_copy(a_hbm.at[row], aidx_v)
            pltpu.sync_copy(b_hbm.at[row], bidx_v)
            cpa = pltpu.async_copy(z_hbm.at[aidx_v], za_v, sema)
            cpb = pltpu.async_copy(z_hbm.at[bidx_v], zb_v, semb)
            cpa.wait()
            cpb.wait()
            for i in range(B // L):
                rows16 = lanes + (i * L)
                acc = _zero16f()
                for k in range(D):
                    cols = jnp.full((L,), k, jnp.int32)
                    va = plsc.load_gather(za_v, [rows16, cols])
                    vb = plsc.load_gather(zb_v, [rows16, cols])
                    acc = acc + va * vb
                scr_v[pl.ds(i * L, L)] = acc
            pltpu.sync_copy(scr_v, out_hbm.at[pl.ds(row * B, B)])
            return 0

        lax.fori_loop(0, RPT, _batch, 0)

    _polarity(pa_hbm, pb_hbm, pos_out)
    _polarity(na_hbm, nb_hbm, neg_out)


BN = 5000


def _tc_a_body(dp_ref, x_ref, w1_ref, dinv_ref, g1_ref):
    deg = jnp.sum(dp_ref[...], axis=1) + 1.0
    dinv = lax.rsqrt(deg)
    dinv_ref[...] = dinv[:, None]
    g1_ref[...] = dinv[:, None] * jnp.dot(
        x_ref[...], w1_ref[...], preferred_element_type=jnp.float32,
        precision=lax.Precision.HIGHEST)


_tc_a = pl.pallas_call(
    _tc_a_body,
    grid=(N // BN,),
    in_specs=[
        pl.BlockSpec((BN, NC), lambda i: (i, 0)),
        pl.BlockSpec((BN, D), lambda i: (i, 0)),
        pl.BlockSpec((D, D), lambda i: (0, 0)),
    ],
    out_specs=[
        pl.BlockSpec((BN, 1), lambda i: (i, 0)),
        pl.BlockSpec((BN, D), lambda i: (i, 0)),
    ],
    out_shape=[
        jax.ShapeDtypeStruct((N, 1), jnp.float32),
        jax.ShapeDtypeStruct((N, D), jnp.float32),
    ],
)


def _tc_b_body(acc_ref, g1_ref, dinv_ref, b1_ref, w2_ref, g2_ref):
    dinv = dinv_ref[...]
    h = jax.nn.relu(dinv * (acc_ref[...] + g1_ref[...]) + b1_ref[...])
    g2_ref[...] = dinv * jnp.dot(
        h, w2_ref[...], preferred_element_type=jnp.float32,
        precision=lax.Precision.HIGHEST)


_tc_b = pl.pallas_call(
    _tc_b_body,
    grid=(N // BN,),
    in_specs=[
        pl.BlockSpec((BN, D), lambda i: (i, 0)),
        pl.BlockSpec((BN, D), lambda i: (i, 0)),
        pl.BlockSpec((BN, 1), lambda i: (i, 0)),
        pl.BlockSpec((1, D), lambda i: (0, 0)),
        pl.BlockSpec((D, D), lambda i: (0, 0)),
    ],
    out_specs=pl.BlockSpec((BN, D), lambda i: (i, 0)),
    out_shape=jax.ShapeDtypeStruct((N, D), jnp.float32),
)


def _tc_c_body(acc_ref, g2_ref, dinv_ref, b2_ref, z_ref):
    z_ref[...] = (dinv_ref[...] * (acc_ref[...] + g2_ref[...])
                  + b2_ref[...])


_tc_c = pl.pallas_call(
    _tc_c_body,
    grid=(N // BN,),
    in_specs=[
        pl.BlockSpec((BN, D), lambda i: (i, 0)),
        pl.BlockSpec((BN, D), lambda i: (i, 0)),
        pl.BlockSpec((BN, 1), lambda i: (i, 0)),
        pl.BlockSpec((1, D), lambda i: (0, 0)),
    ],
    out_specs=pl.BlockSpec((BN, D), lambda i: (i, 0)),
    out_shape=jax.ShapeDtypeStruct((N, D), jnp.float32),
)


def _pad_idx(a, fill):
    pad = jnp.full((PE - E,), fill, jnp.int32)
    return jnp.concatenate([a, pad]).reshape(ROWS, B)


def kernel(x, edge_index, pos_edge_index, neg_edge_index, W1, b1, W2, b2):
    srcP = _pad_idx(edge_index[0], 0)
    dstP = _pad_idx(edge_index[1], -1)
    paP = _pad_idx(pos_edge_index[0], 0)
    pbP = _pad_idx(pos_edge_index[1], 0)
    naP = _pad_idx(neg_edge_index[0], 0)
    nbP = _pad_idx(neg_edge_index[1], 0)

    deg_p = _deg_kernel(dstP)
    dp = jnp.stack([deg_p[:N], deg_p[NP:NP + N]], axis=1)
    dinv2d, g1 = _tc_a(dp, x, W1)
    acc1 = _conv_kernel(g1, srcP, dstP)
    g2 = _tc_b(acc1, g1, dinv2d, b1.reshape(1, D), W2)
    acc2 = _conv_kernel(g2, srcP, dstP)
    z = _tc_c(acc2, g2, dinv2d, b2.reshape(1, D))
    pos_s, neg_s = _score_kernel(z, paP, pbP, naP, nbP)
    return (pos_s[:E], neg_s[:E])


# speedup vs baseline: 5.3855x; 1.4771x over previous
import functools

import jax
import jax.numpy as jnp
from jax import lax
from jax.experimental import pallas as pl
from jax.experimental.pallas import tpu as pltpu
from jax.experimental.pallas import tpu_sc as plsc

N = 50000
E = 800000
D = 64
NC, NS, L = 2, 16, 16
NW = NC * NS
HALF = N // NC
HALFP = HALF + 8
NP = 50176
B = 128
PE = 819200
ROWS = PE // B
RPT = ROWS // NW
RPS = ROWS // NS

_mesh = plsc.VectorSubcoreMesh(core_axis_name="c", subcore_axis_name="s")


def _zero16f():
    return jnp.zeros((L,), jnp.float32)


NPS = NP // NS


@functools.partial(
    pl.kernel,
    out_type=jax.ShapeDtypeStruct((NC * NP,), jnp.float32),
    mesh=_mesh,
    scratch_types=[
        pltpu.VMEM((NPS,), jnp.float32),
        pltpu.VMEM((B,), jnp.float32),
        pltpu.VMEM((B,), jnp.int32),
        pltpu.VMEM((B,), jnp.int32),
        pltpu.VMEM_SHARED((NP,), jnp.float32),
    ],
)
def _deg_kernel(dst_hbm, deg_out, stage_v, ones_v, didx_v, sidx_v, deg_sh):
    c = lax.axis_index("c")
    s = lax.axis_index("s")
    wid = s * NC + c

    one16 = jnp.ones((L,), jnp.float32)
    for k in range(B // L):
        ones_v[pl.ds(k * L, L)] = one16

    def _zero(i, _):
        stage_v[pl.ds(i * L, L)] = _zero16f()
        return 0

    lax.fori_loop(0, NPS // L, _zero, 0)
    pltpu.sync_copy(stage_v, deg_sh.at[pl.ds(s * NPS, NPS)])
    plsc.subcore_barrier()

    row0 = wid * RPT
    trash = jnp.full((L,), N, jnp.int32)

    def _batch(b, _):
        pltpu.sync_copy(dst_hbm.at[row0 + b], didx_v)
        for k in range(B // L):
            dv = didx_v[pl.ds(k * L, L)]
            sidx_v[pl.ds(k * L, L)] = jnp.where(dv >= 0, dv, trash)
        pltpu.sync_copy(ones_v, deg_sh.at[sidx_v], add=True)
        return 0

    lax.fori_loop(0, RPT, _batch, 0)
    plsc.subcore_barrier()

    pltpu.sync_copy(deg_sh.at[pl.ds(s * NPS, NPS)], stage_v)
    pltpu.sync_copy(stage_v, deg_out.at[pl.ds(c * NP + s * NPS, NPS)])


EPT = RPS * B
CH = 2048
CB = CH // B
NCHUNK = EPT // CH


@functools.partial(
    pl.kernel,
    out_type=jax.ShapeDtypeStruct((N, D), jnp.float32),
    mesh=_mesh,
    compiler_params=pltpu.CompilerParams(use_tc_tiling_on_sc=False,
                                         needs_layout_passes=False),
    scratch_types=[
        pltpu.VMEM((CH,), jnp.int32),
        pltpu.VMEM((CH,), jnp.int32),
        pltpu.VMEM((CH,), jnp.int32),
        pltpu.VMEM((CH,), jnp.int32),
        pltpu.VMEM((B, D), jnp.float32),
        pltpu.VMEM((B, D), jnp.float32),
        pltpu.VMEM((B,), jnp.int32),
        pltpu.VMEM((B,), jnp.int32),
        pltpu.VMEM_SHARED((HALFP, D), jnp.float32),
        pltpu.SemaphoreType.DMA,
        pltpu.SemaphoreType.DMA,
        pltpu.SemaphoreType.DMA,
        pltpu.SemaphoreType.DMA,
        pltpu.SemaphoreType.DMA,
        pltpu.SemaphoreType.DMA,
    ],
)
def _conv_kernel(g_hbm, src_hbm, dst_hbm, acc_out, gixA_v, gixB_v, dixA_v,
                 dixB_v, rows0_v, rows1_v, sidx0_v, sidx1_v, acc_sh,
                 semg0, semg1, sems0, sems1, semiA, semiB):
    c = lax.axis_index("c")
    s = lax.axis_index("s")
    e0 = s * EPT
    lo = c * HALF

    def _zrow(i, _):
        for k in range(D // L):
            rows0_v[i, pl.ds(k * L, L)] = _zero16f()
        return 0

    lax.fori_loop(0, B, _zrow, 0)

    zr0 = s * (HALFP // NS)

    def _zacc(i, _):
        pltpu.sync_copy(rows0_v, acc_sh.at[pl.ds(zr0 + i * B, B)])
        return 0

    lax.fori_loop(0, 12, _zacc, 0)
    pltpu.sync_copy(rows0_v.at[pl.ds(0, 27)],
                    acc_sh.at[pl.ds(zr0 + 12 * B, 27)])

    pltpu.sync_copy(src_hbm.at[pl.ds(e0, CH)], gixA_v)
    pltpu.sync_copy(dst_hbm.at[pl.ds(e0, CH)], dixA_v)
    plsc.subcore_barrier()

    def _issue_gather(gix_v, lb, rows_v, semg):
        pltpu.async_copy(g_hbm.at[gix_v.at[pl.ds(lb * B, B)]], rows_v, semg)

    def _chunk(q, gix_v, dix_v, gix_o, dix_o, semi_v, semi_o):
        @pl.when(q > 0)
        def _():
            pltpu.make_async_copy(src_hbm.at[pl.ds(e0, CH)], gix_v,
                                  semi_v).wait()
            pltpu.make_async_copy(dst_hbm.at[pl.ds(e0, CH)], dix_v,
                                  semi_v).wait()

        @pl.when(q + 1 < NCHUNK)
        def _():
            nb = e0 + (q + 1) * CH
            pltpu.async_copy(src_hbm.at[pl.ds(nb, CH)], gix_o, semi_o)
            pltpu.async_copy(dst_hbm.at[pl.ds(nb, CH)], dix_o, semi_o)

        trash = jnp.full((L,), HALF, jnp.int32)

        def _remap(t, _):
            dv = dix_v[pl.ds(t * L, L)]
            loc = dv - lo
            ok = (loc >= 0) & (loc < HALF)
            dix_v[pl.ds(t * L, L)] = jnp.where(ok, loc, trash)
            return 0

        lax.fori_loop(0, CH // L, _remap, 0)

        def _half(lb, rows_v, sidx_v, semg, sems, rows_po, sidx_po, semg_o,
                  sems_o):
            @pl.when(lb > 0)
            def _():
                pltpu.make_async_copy(rows_po, acc_sh.at[sidx_po],
                                      sems_o).wait()

            @pl.when(lb + 1 < CB)
            def _():
                _issue_gather(gix_v, lb + 1, rows_po, semg_o)

            pltpu.make_async_copy(g_hbm.at[gix_v.at[pl.ds(lb * B, B)]],
                                  rows_v, semg).wait()
            for k in range(B // L):
                sidx_v[pl.ds(k * L, L)] = dix_v[pl.ds(lb * B + k * L, L)]
            pltpu.async_copy(rows_v, acc_sh.at[sidx_v], sems, add=True)

        _issue_gather(gix_v, 0, rows0_v, semg0)

        def _pair(p, _):
            _half(2 * p, rows0_v, sidx0_v, semg0, sems0,
                  rows1_v, sidx1_v, semg1, sems1)
            _half(2 * p + 1, rows1_v, sidx1_v, semg1, sems1,
                  rows0_v, sidx0_v, semg0, sems0)
            return 0

        lax.fori_loop(0, CB // 2, _pair, 0)
        pltpu.make_async_copy(rows1_v, acc_sh.at[sidx1_v], sems1).wait()

    def _cpair(p, _):
        _chunk(2 * p, gixA_v, dixA_v, gixB_v, dixB_v, semiA, semiB)
        _chunk(2 * p + 1, gixB_v, dixB_v, gixA_v, dixA_v, semiB, semiA)
        return 0

    lax.fori_loop(0, NCHUNK // 2, _cpair, 0)
    _chunk(NCHUNK - 1, gixA_v, dixA_v, gixB_v, dixB_v, semiA, semiB)
    plsc.subcore_barrier()

    out0 = c * HALF
    cr0 = s * 1568

    @pl.when(s < NS - 1)
    def _():
        pltpu.sync_copy(acc_sh.at[pl.ds(cr0, 1568)],
                        acc_out.at[pl.ds(out0 + cr0, 1568)])

    @pl.when(s == NS - 1)
    def _():
        pltpu.sync_copy(acc_sh.at[pl.ds(cr0, 1480)],
                        acc_out.at[pl.ds(out0 + cr0, 1480)])


EPQ = RPT * B
SBLK = 8


@functools.partial(
    pl.kernel,
    out_type=(jax.ShapeDtypeStruct((PE,), jnp.float32),
              jax.ShapeDtypeStruct((PE,), jnp.float32)),
    mesh=_mesh,
    compiler_params=pltpu.CompilerParams(use_tc_tiling_on_sc=False,
                                         needs_layout_passes=False),
    scratch_types=[
        pltpu.VMEM((EPQ,), jnp.int32),
        pltpu.VMEM((EPQ,), jnp.int32),
        pltpu.VMEM((B, D), jnp.float32),
        pltpu.VMEM((B, D), jnp.float32),
        pltpu.VMEM((B, D), jnp.float32),
        pltpu.VMEM((B, D), jnp.float32),
        pltpu.VMEM((SBLK * B,), jnp.float32),
        pltpu.SemaphoreType.DMA,
        pltpu.SemaphoreType.DMA,
        pltpu.SemaphoreType.DMA,
        pltpu.SemaphoreType.DMA,
    ],
)
def _score_kernel(z_hbm, pa_hbm, pb_hbm, na_hbm, nb_hbm, pos_out, neg_out,
                  aidx_v, bidx_v, za0_v, za1_v, zb0_v, zb1_v, scr_v,
                  sa0, sa1, sb0, sb1):
    c = lax.axis_index("c")
    s = lax.axis_index("s")
    wid = s * NC + c
    e0 = wid * EPQ
    lanes = lax.iota(jnp.int32, L)

    def _polarity(a_hbm, b_hbm, out_hbm):
        pltpu.sync_copy(a_hbm.at[pl.ds(e0, EPQ)], aidx_v)
        pltpu.sync_copy(b_hbm.at[pl.ds(e0, EPQ)], bidx_v)

        def _issue(b, za_v, zb_v, sa, sb):
            pltpu.async_copy(z_hbm.at[aidx_v.at[pl.ds(b * B, B)]], za_v, sa)
            pltpu.async_copy(z_hbm.at[bidx_v.at[pl.ds(b * B, B)]], zb_v, sb)

        def _half(b, j, za_v, zb_v, sa, sb):
            pltpu.make_async_copy(z_hbm.at[aidx_v.at[pl.ds(b * B, B)]],
                                  za_v, sa).wait()
            pltpu.make_async_copy(z_hbm.at[bidx_v.at[pl.ds(b * B, B)]],
                                  zb_v, sb).wait()

            def _sub(i, _):
                rows16 = lanes + i * L
                acc = _zero16f()
                for k in range(D):
                    cols = jnp.full((L,), k, jnp.int32)
                    va = plsc.load_gather(za_v, [rows16, cols])
                    vb = plsc.load_gather(zb_v, [rows16, cols])
                    acc = acc + va * vb
                scr_v[pl.ds((j * (B // L) + i) * L, L)] = acc
                return 0

            lax.fori_loop(0, B // L, _sub, 0)

            @pl.when(b + 2 < RPT)
            def _():
                _issue(b + 2, za_v, zb_v, sa, sb)

        def _blk(t, _):
            b0 = t * SBLK

            def _pair(p, _):
                j = 2 * p
                _half(b0 + j, j, za0_v, zb0_v, sa0, sb0)
                _half(b0 + j + 1, j + 1, za1_v, zb1_v, sa1, sb1)
                return 0

            lax.fori_loop(0, SBLK // 2, _pair, 0)
            pltpu.sync_copy(scr_v,
                            out_hbm.at[pl.ds(e0 + b0 * B, SBLK * B)])
            return 0

        _issue(0, za0_v, zb0_v, sa0, sb0)
        _issue(1, za1_v, zb1_v, sa1, sb1)
        lax.fori_loop(0, RPT // SBLK, _blk, 0)

    _polarity(pa_hbm, pb_hbm, pos_out)
    _polarity(na_hbm, nb_hbm, neg_out)


BN = 5000


def _tc_a_body(dp_ref, x_ref, w1_ref, dinv_ref, g1_ref):
    deg = jnp.sum(dp_ref[...], axis=1) + 1.0
    dinv = lax.rsqrt(deg)
    dinv_ref[...] = dinv[:, None]
    g1_ref[...] = dinv[:, None] * jnp.dot(
        x_ref[...], w1_ref[...], preferred_element_type=jnp.float32,
        precision=lax.Precision.HIGHEST)


_tc_a = pl.pallas_call(
    _tc_a_body,
    grid=(N // BN,),
    in_specs=[
        pl.BlockSpec((BN, NC), lambda i: (i, 0)),
        pl.BlockSpec((BN, D), lambda i: (i, 0)),
        pl.BlockSpec((D, D), lambda i: (0, 0)),
    ],
    out_specs=[
        pl.BlockSpec((BN, 1), lambda i: (i, 0)),
        pl.BlockSpec((BN, D), lambda i: (i, 0)),
    ],
    out_shape=[
        jax.ShapeDtypeStruct((N, 1), jnp.float32),
        jax.ShapeDtypeStruct((N, D), jnp.float32),
    ],
)


def _tc_b_body(acc_ref, g1_ref, dinv_ref, b1_ref, w2_ref, g2_ref):
    dinv = dinv_ref[...]
    h = jax.nn.relu(dinv * (acc_ref[...] + g1_ref[...]) + b1_ref[...])
    g2_ref[...] = dinv * jnp.dot(
        h, w2_ref[...], preferred_element_type=jnp.float32,
        precision=lax.Precision.HIGHEST)


_tc_b = pl.pallas_call(
    _tc_b_body,
    grid=(N // BN,),
    in_specs=[
        pl.BlockSpec((BN, D), lambda i: (i, 0)),
        pl.BlockSpec((BN, D), lambda i: (i, 0)),
        pl.BlockSpec((BN, 1), lambda i: (i, 0)),
        pl.BlockSpec((1, D), lambda i: (0, 0)),
        pl.BlockSpec((D, D), lambda i: (0, 0)),
    ],
    out_specs=pl.BlockSpec((BN, D), lambda i: (i, 0)),
    out_shape=jax.ShapeDtypeStruct((N, D), jnp.float32),
)


def _tc_c_body(acc_ref, g2_ref, dinv_ref, b2_ref, z_ref):
    z_ref[...] = (dinv_ref[...] * (acc_ref[...] + g2_ref[...])
                  + b2_ref[...])


_tc_c = pl.pallas_call(
    _tc_c_body,
    grid=(N // BN,),
    in_specs=[
        pl.BlockSpec((BN, D), lambda i: (i, 0)),
        pl.BlockSpec((BN, D), lambda i: (i, 0)),
        pl.BlockSpec((BN, 1), lambda i: (i, 0)),
        pl.BlockSpec((1, D), lambda i: (0, 0)),
    ],
    out_specs=pl.BlockSpec((BN, D), lambda i: (i, 0)),
    out_shape=jax.ShapeDtypeStruct((N, D), jnp.float32),
)


def _pad_idx(a, fill):
    pad = jnp.full((PE - E,), fill, jnp.int32)
    return jnp.concatenate([a, pad]).reshape(ROWS, B)


def kernel(x, edge_index, pos_edge_index, neg_edge_index, W1, b1, W2, b2):
    dstP = _pad_idx(edge_index[1], -1)
    srcF = _pad_idx(edge_index[0], 0).reshape(PE)
    dstF = dstP.reshape(PE)
    paF = _pad_idx(pos_edge_index[0], 0).reshape(PE)
    pbF = _pad_idx(pos_edge_index[1], 0).reshape(PE)
    naF = _pad_idx(neg_edge_index[0], 0).reshape(PE)
    nbF = _pad_idx(neg_edge_index[1], 0).reshape(PE)

    deg_p = _deg_kernel(dstP)
    dp = jnp.stack([deg_p[:N], deg_p[NP:NP + N]], axis=1)
    dinv2d, g1 = _tc_a(dp, x, W1)
    acc1 = _conv_kernel(g1, srcF, dstF)
    g2 = _tc_b(acc1, g1, dinv2d, b1.reshape(1, D), W2)
    acc2 = _conv_kernel(g2, srcF, dstF)
    z = _tc_c(acc2, g2, dinv2d, b2.reshape(1, D))
    pos_s, neg_s = _score_kernel(z, paF, pbF, naF, nbF)
    return (pos_s[:E], neg_s[:E])


# speedup vs baseline: 6.9210x; 1.2851x over previous
import functools

import jax
import jax.numpy as jnp
from jax import lax
from jax.experimental import pallas as pl
from jax.experimental.pallas import tpu as pltpu
from jax.experimental.pallas import tpu_sc as plsc

N = 50000
E = 800000
D = 64
NC, NS, L = 2, 16, 16
NW = NC * NS
HALF = N // NC
HALFP = HALF + 8
NP = 50176
B = 128
PE = 819200
ROWS = PE // B
RPT = ROWS // NW
RPS = ROWS // NS

_mesh = plsc.VectorSubcoreMesh(core_axis_name="c", subcore_axis_name="s")


def _zero16f():
    return jnp.zeros((L,), jnp.float32)


NPS = NP // NS


@functools.partial(
    pl.kernel,
    out_type=jax.ShapeDtypeStruct((NC * NP,), jnp.float32),
    mesh=_mesh,
    scratch_types=[
        pltpu.VMEM((NPS,), jnp.float32),
        pltpu.VMEM((B,), jnp.float32),
        pltpu.VMEM((B,), jnp.int32),
        pltpu.VMEM((B,), jnp.int32),
        pltpu.VMEM_SHARED((NP,), jnp.float32),
    ],
)
def _deg_kernel(dst_hbm, deg_out, stage_v, ones_v, didx_v, sidx_v, deg_sh):
    c = lax.axis_index("c")
    s = lax.axis_index("s")
    wid = s * NC + c

    one16 = jnp.ones((L,), jnp.float32)
    for k in range(B // L):
        ones_v[pl.ds(k * L, L)] = one16

    def _zero(i, _):
        stage_v[pl.ds(i * L, L)] = _zero16f()
        return 0

    lax.fori_loop(0, NPS // L, _zero, 0)
    pltpu.sync_copy(stage_v, deg_sh.at[pl.ds(s * NPS, NPS)])
    plsc.subcore_barrier()

    row0 = wid * RPT
    trash = jnp.full((L,), N, jnp.int32)

    def _batch(b, _):
        pltpu.sync_copy(dst_hbm.at[row0 + b], didx_v)
        for k in range(B // L):
            dv = didx_v[pl.ds(k * L, L)]
            sidx_v[pl.ds(k * L, L)] = jnp.where(dv >= 0, dv, trash)
        pltpu.sync_copy(ones_v, deg_sh.at[sidx_v], add=True)
        return 0

    lax.fori_loop(0, RPT, _batch, 0)
    plsc.subcore_barrier()

    pltpu.sync_copy(deg_sh.at[pl.ds(s * NPS, NPS)], stage_v)
    pltpu.sync_copy(stage_v, deg_out.at[pl.ds(c * NP + s * NPS, NPS)])


EPT = RPS * B
CH = 2048
CB = CH // B
NCHUNK = EPT // CH


@functools.partial(
    pl.kernel,
    out_type=jax.ShapeDtypeStruct((N, D), jnp.float32),
    mesh=_mesh,
    compiler_params=pltpu.CompilerParams(use_tc_tiling_on_sc=False,
                                         needs_layout_passes=False),
    scratch_types=[
        pltpu.VMEM((CH,), jnp.int32),
        pltpu.VMEM((CH,), jnp.int32),
        pltpu.VMEM((CH,), jnp.int32),
        pltpu.VMEM((CH,), jnp.int32),
        pltpu.VMEM((B, D), jnp.float32),
        pltpu.VMEM((B, D), jnp.float32),
        pltpu.VMEM((B,), jnp.int32),
        pltpu.VMEM((B,), jnp.int32),
        pltpu.VMEM_SHARED((HALFP, D), jnp.float32),
        pltpu.SemaphoreType.DMA,
        pltpu.SemaphoreType.DMA,
        pltpu.SemaphoreType.DMA,
        pltpu.SemaphoreType.DMA,
        pltpu.SemaphoreType.DMA,
        pltpu.SemaphoreType.DMA,
    ],
)
def _conv_kernel(g_hbm, src_hbm, dst_hbm, acc_out, gixA_v, gixB_v, dixA_v,
                 dixB_v, rows0_v, rows1_v, sidx0_v, sidx1_v, acc_sh,
                 semg0, semg1, sems0, sems1, semiA, semiB):
    c = lax.axis_index("c")
    s = lax.axis_index("s")
    e0 = s * EPT
    lo = c * HALF

    def _zrow(i, _):
        for k in range(D // L):
            rows0_v[i, pl.ds(k * L, L)] = _zero16f()
        return 0

    lax.fori_loop(0, B, _zrow, 0)

    zr0 = s * (HALFP // NS)

    def _zacc(i, _):
        pltpu.sync_copy(rows0_v, acc_sh.at[pl.ds(zr0 + i * B, B)])
        return 0

    lax.fori_loop(0, 12, _zacc, 0)
    pltpu.sync_copy(rows0_v.at[pl.ds(0, 27)],
                    acc_sh.at[pl.ds(zr0 + 12 * B, 27)])

    pltpu.sync_copy(src_hbm.at[pl.ds(e0, CH)], gixA_v)
    pltpu.sync_copy(dst_hbm.at[pl.ds(e0, CH)], dixA_v)
    plsc.subcore_barrier()

    def _issue_gather(gix_v, lb, rows_v, semg):
        pltpu.async_copy(g_hbm.at[gix_v.at[pl.ds(lb * B, B)]], rows_v, semg)

    def _chunk(q, gix_v, dix_v, gix_o, dix_o, semi_v, semi_o):
        @pl.when(q > 0)
        def _():
            pltpu.make_async_copy(src_hbm.at[pl.ds(e0, CH)], gix_v,
                                  semi_v).wait()
            pltpu.make_async_copy(dst_hbm.at[pl.ds(e0, CH)], dix_v,
                                  semi_v).wait()

        @pl.when(q + 1 < NCHUNK)
        def _():
            nb = e0 + (q + 1) * CH
            pltpu.async_copy(src_hbm.at[pl.ds(nb, CH)], gix_o, semi_o)
            pltpu.async_copy(dst_hbm.at[pl.ds(nb, CH)], dix_o, semi_o)

        trash = jnp.full((L,), HALF, jnp.int32)

        def _remap(t, _):
            dv = dix_v[pl.ds(t * L, L)]
            loc = dv - lo
            ok = (loc >= 0) & (loc < HALF)
            dix_v[pl.ds(t * L, L)] = jnp.where(ok, loc, trash)
            return 0

        lax.fori_loop(0, CH // L, _remap, 0)

        def _half(lb, rows_v, sidx_v, semg, sems, rows_po, sidx_po, semg_o,
                  sems_o):
            @pl.when(lb > 0)
            def _():
                pltpu.make_async_copy(rows_po, acc_sh.at[sidx_po],
                                      sems_o).wait()

            @pl.when(lb + 1 < CB)
            def _():
                _issue_gather(gix_v, lb + 1, rows_po, semg_o)

            pltpu.make_async_copy(g_hbm.at[gix_v.at[pl.ds(lb * B, B)]],
                                  rows_v, semg).wait()
            for k in range(B // L):
                sidx_v[pl.ds(k * L, L)] = dix_v[pl.ds(lb * B + k * L, L)]
            pltpu.async_copy(rows_v, acc_sh.at[sidx_v], sems, add=True)

        _issue_gather(gix_v, 0, rows0_v, semg0)

        def _pair(p, _):
            _half(2 * p, rows0_v, sidx0_v, semg0, sems0,
                  rows1_v, sidx1_v, semg1, sems1)
            _half(2 * p + 1, rows1_v, sidx1_v, semg1, sems1,
                  rows0_v, sidx0_v, semg0, sems0)
            return 0

        lax.fori_loop(0, CB // 2, _pair, 0)
        pltpu.make_async_copy(rows1_v, acc_sh.at[sidx1_v], sems1).wait()

    def _cpair(p, _):
        _chunk(2 * p, gixA_v, dixA_v, gixB_v, dixB_v, semiA, semiB)
        _chunk(2 * p + 1, gixB_v, dixB_v, gixA_v, dixA_v, semiB, semiA)
        return 0

    lax.fori_loop(0, NCHUNK // 2, _cpair, 0)
    _chunk(NCHUNK - 1, gixA_v, dixA_v, gixB_v, dixB_v, semiA, semiB)
    plsc.subcore_barrier()

    out0 = c * HALF
    cr0 = s * 1568

    @pl.when(s < NS - 1)
    def _():
        pltpu.sync_copy(acc_sh.at[pl.ds(cr0, 1568)],
                        acc_out.at[pl.ds(out0 + cr0, 1568)])

    @pl.when(s == NS - 1)
    def _():
        pltpu.sync_copy(acc_sh.at[pl.ds(cr0, 1480)],
                        acc_out.at[pl.ds(out0 + cr0, 1480)])


EPQ = RPT * B
SBLK = 8


@functools.partial(
    pl.kernel,
    out_type=(jax.ShapeDtypeStruct((PE,), jnp.float32),
              jax.ShapeDtypeStruct((PE,), jnp.float32)),
    mesh=_mesh,
    compiler_params=pltpu.CompilerParams(use_tc_tiling_on_sc=False,
                                         needs_layout_passes=False),
    scratch_types=[
        pltpu.VMEM((EPQ,), jnp.int32),
        pltpu.VMEM((EPQ,), jnp.int32),
        pltpu.VMEM((B, D), jnp.float32),
        pltpu.VMEM((B, D), jnp.float32),
        pltpu.VMEM((B, D), jnp.float32),
        pltpu.VMEM((B, D), jnp.float32),
        pltpu.VMEM((L, L + 1), jnp.float32),
        pltpu.VMEM((SBLK * B,), jnp.float32),
        pltpu.SemaphoreType.DMA,
        pltpu.SemaphoreType.DMA,
        pltpu.SemaphoreType.DMA,
        pltpu.SemaphoreType.DMA,
    ],
)
def _score_kernel(z_hbm, pa_hbm, pb_hbm, na_hbm, nb_hbm, pos_out, neg_out,
                  aidx_v, bidx_v, za0_v, za1_v, zb0_v, zb1_v, t2d_v, scr_v,
                  sa0, sa1, sb0, sb1):
    c = lax.axis_index("c")
    s = lax.axis_index("s")
    wid = s * NC + c
    e0 = wid * EPQ
    lanes = lax.iota(jnp.int32, L)

    def _polarity(a_hbm, b_hbm, out_hbm):
        pltpu.sync_copy(a_hbm.at[pl.ds(e0, EPQ)], aidx_v)
        pltpu.sync_copy(b_hbm.at[pl.ds(e0, EPQ)], bidx_v)

        def _issue(b, za_v, zb_v, sa, sb):
            pltpu.async_copy(z_hbm.at[aidx_v.at[pl.ds(b * B, B)]], za_v, sa)
            pltpu.async_copy(z_hbm.at[bidx_v.at[pl.ds(b * B, B)]], zb_v, sb)

        def _half(b, j, za_v, zb_v, sa, sb):
            pltpu.make_async_copy(z_hbm.at[aidx_v.at[pl.ds(b * B, B)]],
                                  za_v, sa).wait()
            pltpu.make_async_copy(z_hbm.at[bidx_v.at[pl.ds(b * B, B)]],
                                  zb_v, sb).wait()

            def _sub(i, _):
                for e in range(L):
                    row = jnp.full((L,), i * L + e, jnp.int32)
                    p = _zero16f()
                    for k in range(D // L):
                        cols = lanes + k * L
                        va = plsc.load_gather(za_v, [row, cols])
                        vb = plsc.load_gather(zb_v, [row, cols])
                        p = p + va * vb
                    t2d_v[e, pl.ds(0, L)] = p
                a0 = _zero16f()
                a1 = _zero16f()
                a2 = _zero16f()
                a3 = _zero16f()
                for k in range(L // 4):
                    c0 = jnp.full((L,), 4 * k, jnp.int32)
                    c1 = jnp.full((L,), 4 * k + 1, jnp.int32)
                    c2 = jnp.full((L,), 4 * k + 2, jnp.int32)
                    c3 = jnp.full((L,), 4 * k + 3, jnp.int32)
                    a0 = a0 + plsc.load_gather(t2d_v, [lanes, c0])
                    a1 = a1 + plsc.load_gather(t2d_v, [lanes, c1])
                    a2 = a2 + plsc.load_gather(t2d_v, [lanes, c2])
                    a3 = a3 + plsc.load_gather(t2d_v, [lanes, c3])
                acc = (a0 + a1) + (a2 + a3)
                scr_v[pl.ds((j * (B // L) + i) * L, L)] = acc
                return 0

            lax.fori_loop(0, B // L, _sub, 0)

            @pl.when(b + 2 < RPT)
            def _():
                _issue(b + 2, za_v, zb_v, sa, sb)

        def _blk(t, _):
            b0 = t * SBLK

            def _pair(p, _):
                j = 2 * p
                _half(b0 + j, j, za0_v, zb0_v, sa0, sb0)
                _half(b0 + j + 1, j + 1, za1_v, zb1_v, sa1, sb1)
                return 0

            lax.fori_loop(0, SBLK // 2, _pair, 0)
            pltpu.sync_copy(scr_v,
                            out_hbm.at[pl.ds(e0 + b0 * B, SBLK * B)])
            return 0

        _issue(0, za0_v, zb0_v, sa0, sb0)
        _issue(1, za1_v, zb1_v, sa1, sb1)
        lax.fori_loop(0, RPT // SBLK, _blk, 0)

    _polarity(pa_hbm, pb_hbm, pos_out)
    _polarity(na_hbm, nb_hbm, neg_out)


BN = 5000


def _tc_a_body(dp_ref, x_ref, w1_ref, dinv_ref, g1_ref):
    deg = jnp.sum(dp_ref[...], axis=1) + 1.0
    dinv = lax.rsqrt(deg)
    dinv_ref[...] = dinv[:, None]
    g1_ref[...] = dinv[:, None] * jnp.dot(
        x_ref[...], w1_ref[...], preferred_element_type=jnp.float32,
        precision=lax.Precision.HIGHEST)


_tc_a = pl.pallas_call(
    _tc_a_body,
    grid=(N // BN,),
    in_specs=[
        pl.BlockSpec((BN, NC), lambda i: (i, 0)),
        pl.BlockSpec((BN, D), lambda i: (i, 0)),
        pl.BlockSpec((D, D), lambda i: (0, 0)),
    ],
    out_specs=[
        pl.BlockSpec((BN, 1), lambda i: (i, 0)),
        pl.BlockSpec((BN, D), lambda i: (i, 0)),
    ],
    out_shape=[
        jax.ShapeDtypeStruct((N, 1), jnp.float32),
        jax.ShapeDtypeStruct((N, D), jnp.float32),
    ],
)


def _tc_b_body(acc_ref, g1_ref, dinv_ref, b1_ref, w2_ref, g2_ref):
    dinv = dinv_ref[...]
    h = jax.nn.relu(dinv * (acc_ref[...] + g1_ref[...]) + b1_ref[...])
    g2_ref[...] = dinv * jnp.dot(
        h, w2_ref[...], preferred_element_type=jnp.float32,
        precision=lax.Precision.HIGHEST)


_tc_b = pl.pallas_call(
    _tc_b_body,
    grid=(N // BN,),
    in_specs=[
        pl.BlockSpec((BN, D), lambda i: (i, 0)),
        pl.BlockSpec((BN, D), lambda i: (i, 0)),
        pl.BlockSpec((BN, 1), lambda i: (i, 0)),
        pl.BlockSpec((1, D), lambda i: (0, 0)),
        pl.BlockSpec((D, D), lambda i: (0, 0)),
    ],
    out_specs=pl.BlockSpec((BN, D), lambda i: (i, 0)),
    out_shape=jax.ShapeDtypeStruct((N, D), jnp.float32),
)


def _tc_c_body(acc_ref, g2_ref, dinv_ref, b2_ref, z_ref):
    z_ref[...] = (dinv_ref[...] * (acc_ref[...] + g2_ref[...])
                  + b2_ref[...])


_tc_c = pl.pallas_call(
    _tc_c_body,
    grid=(N // BN,),
    in_specs=[
        pl.BlockSpec((BN, D), lambda i: (i, 0)),
        pl.BlockSpec((BN, D), lambda i: (i, 0)),
        pl.BlockSpec((BN, 1), lambda i: (i, 0)),
        pl.BlockSpec((1, D), lambda i: (0, 0)),
    ],
    out_specs=pl.BlockSpec((BN, D), lambda i: (i, 0)),
    out_shape=jax.ShapeDtypeStruct((N, D), jnp.float32),
)


def _pad_idx(a, fill):
    pad = jnp.full((PE - E,), fill, jnp.int32)
    return jnp.concatenate([a, pad]).reshape(ROWS, B)


def kernel(x, edge_index, pos_edge_index, neg_edge_index, W1, b1, W2, b2):
    dstP = _pad_idx(edge_index[1], -1)
    srcF = _pad_idx(edge_index[0], 0).reshape(PE)
    dstF = dstP.reshape(PE)
    paF = _pad_idx(pos_edge_index[0], 0).reshape(PE)
    pbF = _pad_idx(pos_edge_index[1], 0).reshape(PE)
    naF = _pad_idx(neg_edge_index[0], 0).reshape(PE)
    nbF = _pad_idx(neg_edge_index[1], 0).reshape(PE)

    deg_p = _deg_kernel(dstP)
    dp = jnp.stack([deg_p[:N], deg_p[NP:NP + N]], axis=1)
    dinv2d, g1 = _tc_a(dp, x, W1)
    acc1 = _conv_kernel(g1, srcF, dstF)
    g2 = _tc_b(acc1, g1, dinv2d, b1.reshape(1, D), W2)
    acc2 = _conv_kernel(g2, srcF, dstF)
    z = _tc_c(acc2, g2, dinv2d, b2.reshape(1, D))
    pos_s, neg_s = _score_kernel(z, paF, pbF, naF, nbF)
    return (pos_s[:E], neg_s[:E])


# speedup vs baseline: 8.0582x; 1.1643x over previous
import functools

import jax
import jax.numpy as jnp
from jax import lax
from jax.experimental import pallas as pl
from jax.experimental.pallas import tpu as pltpu
from jax.experimental.pallas import tpu_sc as plsc

N = 50000
E = 800000
D = 64
NC, NS, L = 2, 16, 16
NW = NC * NS
HALF = N // NC
HALFP = HALF + 8
NP = 50176
B = 128
PE = 819200
ROWS = PE // B
RPT = ROWS // NW
RPS = ROWS // NS

_mesh = plsc.VectorSubcoreMesh(core_axis_name="c", subcore_axis_name="s")


def _zero16f():
    return jnp.zeros((L,), jnp.float32)


NPS = NP // NS


@functools.partial(
    pl.kernel,
    out_type=jax.ShapeDtypeStruct((NC * NP,), jnp.float32),
    mesh=_mesh,
    scratch_types=[
        pltpu.VMEM((NPS,), jnp.float32),
        pltpu.VMEM((B,), jnp.float32),
        pltpu.VMEM((B,), jnp.int32),
        pltpu.VMEM((B,), jnp.int32),
        pltpu.VMEM_SHARED((NP,), jnp.float32),
    ],
)
def _deg_kernel(dst_hbm, deg_out, stage_v, ones_v, didx_v, sidx_v, deg_sh):
    c = lax.axis_index("c")
    s = lax.axis_index("s")
    wid = s * NC + c

    one16 = jnp.ones((L,), jnp.float32)
    for k in range(B // L):
        ones_v[pl.ds(k * L, L)] = one16

    def _zero(i, _):
        stage_v[pl.ds(i * L, L)] = _zero16f()
        return 0

    lax.fori_loop(0, NPS // L, _zero, 0)
    pltpu.sync_copy(stage_v, deg_sh.at[pl.ds(s * NPS, NPS)])
    plsc.subcore_barrier()

    row0 = wid * RPT
    trash = jnp.full((L,), N, jnp.int32)

    def _batch(b, _):
        pltpu.sync_copy(dst_hbm.at[row0 + b], didx_v)
        for k in range(B // L):
            dv = didx_v[pl.ds(k * L, L)]
            sidx_v[pl.ds(k * L, L)] = jnp.where(dv >= 0, dv, trash)
        pltpu.sync_copy(ones_v, deg_sh.at[sidx_v], add=True)
        return 0

    lax.fori_loop(0, RPT, _batch, 0)
    plsc.subcore_barrier()

    pltpu.sync_copy(deg_sh.at[pl.ds(s * NPS, NPS)], stage_v)
    pltpu.sync_copy(stage_v, deg_out.at[pl.ds(c * NP + s * NPS, NPS)])


EPT = RPS * B
CH = 2048
CB = CH // B
NCHUNK = EPT // CH


@functools.partial(
    pl.kernel,
    out_type=jax.ShapeDtypeStruct((N, D), jnp.float32),
    mesh=_mesh,
    compiler_params=pltpu.CompilerParams(use_tc_tiling_on_sc=False,
                                         needs_layout_passes=False),
    scratch_types=[
        pltpu.VMEM((CH,), jnp.int32),
        pltpu.VMEM((CH,), jnp.int32),
        pltpu.VMEM((CH,), jnp.int32),
        pltpu.VMEM((CH,), jnp.int32),
        pltpu.VMEM((B, D), jnp.float32),
        pltpu.VMEM((B, D), jnp.float32),
        pltpu.VMEM((B,), jnp.int32),
        pltpu.VMEM((B,), jnp.int32),
        pltpu.VMEM_SHARED((HALFP, D), jnp.float32),
        pltpu.SemaphoreType.DMA,
        pltpu.SemaphoreType.DMA,
        pltpu.SemaphoreType.DMA,
        pltpu.SemaphoreType.DMA,
        pltpu.SemaphoreType.DMA,
        pltpu.SemaphoreType.DMA,
    ],
)
def _conv_kernel(g_hbm, src_hbm, dst_hbm, acc_out, gixA_v, gixB_v, dixA_v,
                 dixB_v, rows0_v, rows1_v, sidx0_v, sidx1_v, acc_sh,
                 semg0, semg1, sems0, sems1, semiA, semiB):
    c = lax.axis_index("c")
    s = lax.axis_index("s")
    e0 = s * EPT
    lo = c * HALF

    def _zrow(i, _):
        for k in range(D // L):
            rows0_v[i, pl.ds(k * L, L)] = _zero16f()
        return 0

    lax.fori_loop(0, B, _zrow, 0)

    zr0 = s * (HALFP // NS)

    def _zacc(i, _):
        pltpu.sync_copy(rows0_v, acc_sh.at[pl.ds(zr0 + i * B, B)])
        return 0

    lax.fori_loop(0, 12, _zacc, 0)
    pltpu.sync_copy(rows0_v.at[pl.ds(0, 27)],
                    acc_sh.at[pl.ds(zr0 + 12 * B, 27)])

    pltpu.sync_copy(src_hbm.at[pl.ds(e0, CH)], gixA_v)
    pltpu.sync_copy(dst_hbm.at[pl.ds(e0, CH)], dixA_v)
    plsc.subcore_barrier()

    def _issue_gather(gix_v, lb, rows_v, semg):
        pltpu.async_copy(g_hbm.at[gix_v.at[pl.ds(lb * B, B)]], rows_v, semg)

    def _chunk(q, gix_v, dix_v, gix_o, dix_o, semi_v, semi_o):
        @pl.when(q > 0)
        def _():
            pltpu.make_async_copy(src_hbm.at[pl.ds(e0, CH)], gix_v,
                                  semi_v).wait()
            pltpu.make_async_copy(dst_hbm.at[pl.ds(e0, CH)], dix_v,
                                  semi_v).wait()

        @pl.when(q + 1 < NCHUNK)
        def _():
            nb = e0 + (q + 1) * CH
            pltpu.async_copy(src_hbm.at[pl.ds(nb, CH)], gix_o, semi_o)
            pltpu.async_copy(dst_hbm.at[pl.ds(nb, CH)], dix_o, semi_o)

        trash = jnp.full((L,), HALF, jnp.int32)

        def _remap(t, _):
            dv = dix_v[pl.ds(t * L, L)]
            loc = dv - lo
            ok = (loc >= 0) & (loc < HALF)
            dix_v[pl.ds(t * L, L)] = jnp.where(ok, loc, trash)
            return 0

        lax.fori_loop(0, CH // L, _remap, 0)

        def _half(lb, rows_v, sidx_v, semg, sems, rows_po, sidx_po, semg_o,
                  sems_o):
            @pl.when(lb > 0)
            def _():
                pltpu.make_async_copy(rows_po, acc_sh.at[sidx_po],
                                      sems_o).wait()

            @pl.when(lb + 1 < CB)
            def _():
                _issue_gather(gix_v, lb + 1, rows_po, semg_o)

            pltpu.make_async_copy(g_hbm.at[gix_v.at[pl.ds(lb * B, B)]],
                                  rows_v, semg).wait()
            for k in range(B // L):
                sidx_v[pl.ds(k * L, L)] = dix_v[pl.ds(lb * B + k * L, L)]
            pltpu.async_copy(rows_v, acc_sh.at[sidx_v], sems, add=True)

        _issue_gather(gix_v, 0, rows0_v, semg0)

        def _pair(p, _):
            _half(2 * p, rows0_v, sidx0_v, semg0, sems0,
                  rows1_v, sidx1_v, semg1, sems1)
            _half(2 * p + 1, rows1_v, sidx1_v, semg1, sems1,
                  rows0_v, sidx0_v, semg0, sems0)
            return 0

        lax.fori_loop(0, CB // 2, _pair, 0)
        pltpu.make_async_copy(rows1_v, acc_sh.at[sidx1_v], sems1).wait()

    def _cpair(p, _):
        _chunk(2 * p, gixA_v, dixA_v, gixB_v, dixB_v, semiA, semiB)
        _chunk(2 * p + 1, gixB_v, dixB_v, gixA_v, dixA_v, semiB, semiA)
        return 0

    lax.fori_loop(0, NCHUNK // 2, _cpair, 0)
    _chunk(NCHUNK - 1, gixA_v, dixA_v, gixB_v, dixB_v, semiA, semiB)
    plsc.subcore_barrier()

    out0 = c * HALF
    cr0 = s * 1568

    @pl.when(s < NS - 1)
    def _():
        pltpu.sync_copy(acc_sh.at[pl.ds(cr0, 1568)],
                        acc_out.at[pl.ds(out0 + cr0, 1568)])

    @pl.when(s == NS - 1)
    def _():
        pltpu.sync_copy(acc_sh.at[pl.ds(cr0, 1480)],
                        acc_out.at[pl.ds(out0 + cr0, 1480)])


EPQ = RPT * B
SBLK = 8


@functools.partial(
    pl.kernel,
    out_type=(jax.ShapeDtypeStruct((PE,), jnp.float32),
              jax.ShapeDtypeStruct((PE,), jnp.float32)),
    mesh=_mesh,
    compiler_params=pltpu.CompilerParams(use_tc_tiling_on_sc=False,
                                         needs_layout_passes=False),
    scratch_types=[
        pltpu.VMEM((EPQ,), jnp.int32),
        pltpu.VMEM((EPQ,), jnp.int32),
        pltpu.VMEM((B, D // 2), jnp.int32),
        pltpu.VMEM((B, D // 2), jnp.int32),
        pltpu.VMEM((B, D // 2), jnp.int32),
        pltpu.VMEM((B, D // 2), jnp.int32),
        pltpu.VMEM((L, L + 1), jnp.float32),
        pltpu.VMEM((SBLK * B,), jnp.float32),
        pltpu.SemaphoreType.DMA,
        pltpu.SemaphoreType.DMA,
        pltpu.SemaphoreType.DMA,
        pltpu.SemaphoreType.DMA,
    ],
)
def _score_kernel(z_hbm, pa_hbm, pb_hbm, na_hbm, nb_hbm, pos_out, neg_out,
                  aidx_v, bidx_v, za0_v, za1_v, zb0_v, zb1_v, t2d_v, scr_v,
                  sa0, sa1, sb0, sb1):
    c = lax.axis_index("c")
    s = lax.axis_index("s")
    wid = s * NC + c
    e0 = wid * EPQ
    lanes = lax.iota(jnp.int32, L)

    def _polarity(a_hbm, b_hbm, out_hbm):
        pltpu.sync_copy(a_hbm.at[pl.ds(e0, EPQ)], aidx_v)
        pltpu.sync_copy(b_hbm.at[pl.ds(e0, EPQ)], bidx_v)

        def _issue(b, za_v, zb_v, sa, sb):
            pltpu.async_copy(z_hbm.at[aidx_v.at[pl.ds(b * B, B)]], za_v, sa)
            pltpu.async_copy(z_hbm.at[bidx_v.at[pl.ds(b * B, B)]], zb_v, sb)

        def _half(b, j, za_v, zb_v, sa, sb):
            pltpu.make_async_copy(z_hbm.at[aidx_v.at[pl.ds(b * B, B)]],
                                  za_v, sa).wait()
            pltpu.make_async_copy(z_hbm.at[bidx_v.at[pl.ds(b * B, B)]],
                                  zb_v, sb).wait()

            def _sub(i, _):
                for e in range(L):
                    row = jnp.full((L,), i * L + e, jnp.int32)
                    p = _zero16f()
                    q = _zero16f()
                    for k in range(D // (2 * L)):
                        cols = lanes + k * L
                        wa = plsc.load_gather(za_v, [row, cols])
                        wb = plsc.load_gather(zb_v, [row, cols])
                        a0, a1 = plsc.unpack(
                            plsc.bitcast(wa, jnp.bfloat16),
                            format=plsc.PackFormat.INTERLEAVED)
                        b0, b1 = plsc.unpack(
                            plsc.bitcast(wb, jnp.bfloat16),
                            format=plsc.PackFormat.INTERLEAVED)
                        p = p + a0 * b0
                        q = q + a1 * b1
                    t2d_v[e, pl.ds(0, L)] = p + q
                a0 = _zero16f()
                a1 = _zero16f()
                a2 = _zero16f()
                a3 = _zero16f()
                for k in range(L // 4):
                    c0 = jnp.full((L,), 4 * k, jnp.int32)
                    c1 = jnp.full((L,), 4 * k + 1, jnp.int32)
                    c2 = jnp.full((L,), 4 * k + 2, jnp.int32)
                    c3 = jnp.full((L,), 4 * k + 3, jnp.int32)
                    a0 = a0 + plsc.load_gather(t2d_v, [lanes, c0])
                    a1 = a1 + plsc.load_gather(t2d_v, [lanes, c1])
                    a2 = a2 + plsc.load_gather(t2d_v, [lanes, c2])
                    a3 = a3 + plsc.load_gather(t2d_v, [lanes, c3])
                acc = (a0 + a1) + (a2 + a3)
                scr_v[pl.ds((j * (B // L) + i) * L, L)] = acc
                return 0

            lax.fori_loop(0, B // L, _sub, 0)

            @pl.when(b + 2 < RPT)
            def _():
                _issue(b + 2, za_v, zb_v, sa, sb)

        def _blk(t, _):
            b0 = t * SBLK

            def _pair(p, _):
                j = 2 * p
                _half(b0 + j, j, za0_v, zb0_v, sa0, sb0)
                _half(b0 + j + 1, j + 1, za1_v, zb1_v, sa1, sb1)
                return 0

            lax.fori_loop(0, SBLK // 2, _pair, 0)
            pltpu.sync_copy(scr_v,
                            out_hbm.at[pl.ds(e0 + b0 * B, SBLK * B)])
            return 0

        _issue(0, za0_v, zb0_v, sa0, sb0)
        _issue(1, za1_v, zb1_v, sa1, sb1)
        lax.fori_loop(0, RPT // SBLK, _blk, 0)

    _polarity(pa_hbm, pb_hbm, pos_out)
    _polarity(na_hbm, nb_hbm, neg_out)


BN = 5000


def _tc_a_body(dp_ref, x_ref, w1_ref, dinv_ref, g1_ref):
    deg = jnp.sum(dp_ref[...], axis=1) + 1.0
    dinv = lax.rsqrt(deg)
    dinv_ref[...] = dinv[:, None]
    g1_ref[...] = dinv[:, None] * jnp.dot(
        x_ref[...], w1_ref[...], preferred_element_type=jnp.float32,
        precision=lax.Precision.HIGHEST)


_tc_a = pl.pallas_call(
    _tc_a_body,
    grid=(N // BN,),
    in_specs=[
        pl.BlockSpec((BN, NC), lambda i: (i, 0)),
        pl.BlockSpec((BN, D), lambda i: (i, 0)),
        pl.BlockSpec((D, D), lambda i: (0, 0)),
    ],
    out_specs=[
        pl.BlockSpec((BN, 1), lambda i: (i, 0)),
        pl.BlockSpec((BN, D), lambda i: (i, 0)),
    ],
    out_shape=[
        jax.ShapeDtypeStruct((N, 1), jnp.float32),
        jax.ShapeDtypeStruct((N, D), jnp.float32),
    ],
)


def _tc_b_body(acc_ref, g1_ref, dinv_ref, b1_ref, w2_ref, g2_ref):
    dinv = dinv_ref[...]
    h = jax.nn.relu(dinv * (acc_ref[...] + g1_ref[...]) + b1_ref[...])
    g2_ref[...] = dinv * jnp.dot(
        h, w2_ref[...], preferred_element_type=jnp.float32,
        precision=lax.Precision.HIGHEST)


_tc_b = pl.pallas_call(
    _tc_b_body,
    grid=(N // BN,),
    in_specs=[
        pl.BlockSpec((BN, D), lambda i: (i, 0)),
        pl.BlockSpec((BN, D), lambda i: (i, 0)),
        pl.BlockSpec((BN, 1), lambda i: (i, 0)),
        pl.BlockSpec((1, D), lambda i: (0, 0)),
        pl.BlockSpec((D, D), lambda i: (0, 0)),
    ],
    out_specs=pl.BlockSpec((BN, D), lambda i: (i, 0)),
    out_shape=jax.ShapeDtypeStruct((N, D), jnp.float32),
)


def _tc_c_body(acc_ref, g2_ref, dinv_ref, b2_ref, z_ref):
    z_ref[...] = (dinv_ref[...] * (acc_ref[...] + g2_ref[...])
                  + b2_ref[...])


_tc_c = pl.pallas_call(
    _tc_c_body,
    grid=(N // BN,),
    in_specs=[
        pl.BlockSpec((BN, D), lambda i: (i, 0)),
        pl.BlockSpec((BN, D), lambda i: (i, 0)),
        pl.BlockSpec((BN, 1), lambda i: (i, 0)),
        pl.BlockSpec((1, D), lambda i: (0, 0)),
    ],
    out_specs=pl.BlockSpec((BN, D), lambda i: (i, 0)),
    out_shape=jax.ShapeDtypeStruct((N, D), jnp.float32),
)


def _pad_idx(a, fill):
    pad = jnp.full((PE - E,), fill, jnp.int32)
    return jnp.concatenate([a, pad]).reshape(ROWS, B)


def kernel(x, edge_index, pos_edge_index, neg_edge_index, W1, b1, W2, b2):
    dstP = _pad_idx(edge_index[1], -1)
    srcF = _pad_idx(edge_index[0], 0).reshape(PE)
    dstF = dstP.reshape(PE)
    paF = _pad_idx(pos_edge_index[0], 0).reshape(PE)
    pbF = _pad_idx(pos_edge_index[1], 0).reshape(PE)
    naF = _pad_idx(neg_edge_index[0], 0).reshape(PE)
    nbF = _pad_idx(neg_edge_index[1], 0).reshape(PE)

    deg_p = _deg_kernel(dstP)
    dp = jnp.stack([deg_p[:N], deg_p[NP:NP + N]], axis=1)
    dinv2d, g1 = _tc_a(dp, x, W1)
    acc1 = _conv_kernel(g1, srcF, dstF)
    g2 = _tc_b(acc1, g1, dinv2d, b1.reshape(1, D), W2)
    acc2 = _conv_kernel(g2, srcF, dstF)
    z = _tc_c(acc2, g2, dinv2d, b2.reshape(1, D))
    z_bits = lax.bitcast_convert_type(
        z.astype(jnp.bfloat16).reshape(N, D // 2, 2), jnp.int32)
    pos_s, neg_s = _score_kernel(z_bits, paF, pbF, naF, nbF)
    return (pos_s[:E], neg_s[:E])


# speedup vs baseline: 8.0713x; 1.0016x over previous
import functools

import jax
import jax.numpy as jnp
from jax import lax
from jax.experimental import pallas as pl
from jax.experimental.pallas import tpu as pltpu
from jax.experimental.pallas import tpu_sc as plsc

N = 50000
E = 800000
D = 64
NC, NS, L = 2, 16, 16
NW = NC * NS
HALF = N // NC
HALFP = HALF + 8
NP = 50176
B = 128
PE = 819200
ROWS = PE // B
RPT = ROWS // NW
RPS = ROWS // NS

_mesh = plsc.VectorSubcoreMesh(core_axis_name="c", subcore_axis_name="s")


def _zero16f():
    return jnp.zeros((L,), jnp.float32)


NPS = NP // NS


@functools.partial(
    pl.kernel,
    out_type=jax.ShapeDtypeStruct((NC * NP,), jnp.float32),
    mesh=_mesh,
    scratch_types=[
        pltpu.VMEM((NPS,), jnp.float32),
        pltpu.VMEM((B,), jnp.float32),
        pltpu.VMEM((B,), jnp.int32),
        pltpu.VMEM((B,), jnp.int32),
        pltpu.VMEM_SHARED((NP,), jnp.float32),
    ],
)
def _deg_kernel(dst_hbm, deg_out, stage_v, ones_v, didx_v, sidx_v, deg_sh):
    c = lax.axis_index("c")
    s = lax.axis_index("s")
    wid = s * NC + c

    one16 = jnp.ones((L,), jnp.float32)
    for k in range(B // L):
        ones_v[pl.ds(k * L, L)] = one16

    def _zero(i, _):
        stage_v[pl.ds(i * L, L)] = _zero16f()
        return 0

    lax.fori_loop(0, NPS // L, _zero, 0)
    pltpu.sync_copy(stage_v, deg_sh.at[pl.ds(s * NPS, NPS)])
    plsc.subcore_barrier()

    row0 = wid * RPT
    trash = jnp.full((L,), N, jnp.int32)

    def _batch(b, _):
        pltpu.sync_copy(dst_hbm.at[row0 + b], didx_v)
        for k in range(B // L):
            dv = didx_v[pl.ds(k * L, L)]
            sidx_v[pl.ds(k * L, L)] = jnp.where(dv >= 0, dv, trash)
        pltpu.sync_copy(ones_v, deg_sh.at[sidx_v], add=True)
        return 0

    lax.fori_loop(0, RPT, _batch, 0)
    plsc.subcore_barrier()

    pltpu.sync_copy(deg_sh.at[pl.ds(s * NPS, NPS)], stage_v)
    pltpu.sync_copy(stage_v, deg_out.at[pl.ds(c * NP + s * NPS, NPS)])


EPT = RPS * B
CH = 2048
CB = CH // B
NCHUNK = EPT // CH


@functools.partial(
    pl.kernel,
    out_type=jax.ShapeDtypeStruct((N, D), jnp.float32),
    mesh=_mesh,
    compiler_params=pltpu.CompilerParams(use_tc_tiling_on_sc=False,
                                         needs_layout_passes=False),
    scratch_types=[
        pltpu.VMEM((CH,), jnp.int32),
        pltpu.VMEM((CH,), jnp.int32),
        pltpu.VMEM((CH,), jnp.int32),
        pltpu.VMEM((CH,), jnp.int32),
        pltpu.VMEM((B, D), jnp.float32),
        pltpu.VMEM((B, D), jnp.float32),
        pltpu.VMEM((B,), jnp.int32),
        pltpu.VMEM((B,), jnp.int32),
        pltpu.VMEM_SHARED((HALFP, D), jnp.float32),
        pltpu.SemaphoreType.DMA,
        pltpu.SemaphoreType.DMA,
        pltpu.SemaphoreType.DMA,
        pltpu.SemaphoreType.DMA,
        pltpu.SemaphoreType.DMA,
        pltpu.SemaphoreType.DMA,
    ],
)
def _conv_kernel(g_hbm, src_hbm, dst_hbm, acc_out, gixA_v, gixB_v, dixA_v,
                 dixB_v, rows0_v, rows1_v, sidx0_v, sidx1_v, acc_sh,
                 semg0, semg1, sems0, sems1, semiA, semiB):
    c = lax.axis_index("c")
    s = lax.axis_index("s")
    e0 = s * EPT
    lo = c * HALF

    def _zrow(i, _):
        for k in range(D // L):
            rows0_v[i, pl.ds(k * L, L)] = _zero16f()
        return 0

    lax.fori_loop(0, B, _zrow, 0)

    zr0 = s * (HALFP // NS)

    def _zacc(i, _):
        pltpu.sync_copy(rows0_v, acc_sh.at[pl.ds(zr0 + i * B, B)])
        return 0

    lax.fori_loop(0, 12, _zacc, 0)
    pltpu.sync_copy(rows0_v.at[pl.ds(0, 27)],
                    acc_sh.at[pl.ds(zr0 + 12 * B, 27)])

    pltpu.sync_copy(src_hbm.at[pl.ds(e0, CH)], gixA_v)
    pltpu.sync_copy(dst_hbm.at[pl.ds(e0, CH)], dixA_v)
    plsc.subcore_barrier()

    def _issue_gather(gix_v, lb, rows_v, semg):
        pltpu.async_copy(g_hbm.at[gix_v.at[pl.ds(lb * B, B)]], rows_v, semg)

    def _chunk(q, gix_v, dix_v, gix_o, dix_o, semi_v, semi_o):
        @pl.when(q > 0)
        def _():
            pltpu.make_async_copy(src_hbm.at[pl.ds(e0, CH)], gix_v,
                                  semi_v).wait()
            pltpu.make_async_copy(dst_hbm.at[pl.ds(e0, CH)], dix_v,
                                  semi_v).wait()

        @pl.when(q + 1 < NCHUNK)
        def _():
            nb = e0 + (q + 1) * CH
            pltpu.async_copy(src_hbm.at[pl.ds(nb, CH)], gix_o, semi_o)
            pltpu.async_copy(dst_hbm.at[pl.ds(nb, CH)], dix_o, semi_o)

        trash = jnp.full((L,), HALF, jnp.int32)

        def _remap(t, _):
            dv = dix_v[pl.ds(t * L, L)]
            loc = dv - lo
            ok = (loc >= 0) & (loc < HALF)
            dix_v[pl.ds(t * L, L)] = jnp.where(ok, loc, trash)
            return 0

        lax.fori_loop(0, CH // L, _remap, 0)

        def _half(lb, rows_v, sidx_v, semg, sems, rows_po, sidx_po, semg_o,
                  sems_o):
            @pl.when(lb > 0)
            def _():
                pltpu.make_async_copy(rows_po, acc_sh.at[sidx_po],
                                      sems_o).wait()

            @pl.when(lb + 1 < CB)
            def _():
                _issue_gather(gix_v, lb + 1, rows_po, semg_o)

            pltpu.make_async_copy(g_hbm.at[gix_v.at[pl.ds(lb * B, B)]],
                                  rows_v, semg).wait()
            for k in range(B // L):
                sidx_v[pl.ds(k * L, L)] = dix_v[pl.ds(lb * B + k * L, L)]
            pltpu.async_copy(rows_v, acc_sh.at[sidx_v], sems, add=True)

        _issue_gather(gix_v, 0, rows0_v, semg0)

        def _pair(p, _):
            _half(2 * p, rows0_v, sidx0_v, semg0, sems0,
                  rows1_v, sidx1_v, semg1, sems1)
            _half(2 * p + 1, rows1_v, sidx1_v, semg1, sems1,
                  rows0_v, sidx0_v, semg0, sems0)
            return 0

        lax.fori_loop(0, CB // 2, _pair, 0)
        pltpu.make_async_copy(rows1_v, acc_sh.at[sidx1_v], sems1).wait()

    def _cpair(p, _):
        _chunk(2 * p, gixA_v, dixA_v, gixB_v, dixB_v, semiA, semiB)
        _chunk(2 * p + 1, gixB_v, dixB_v, gixA_v, dixA_v, semiB, semiA)
        return 0

    lax.fori_loop(0, NCHUNK // 2, _cpair, 0)
    _chunk(NCHUNK - 1, gixA_v, dixA_v, gixB_v, dixB_v, semiA, semiB)
    plsc.subcore_barrier()

    out0 = c * HALF
    cr0 = s * 1568

    @pl.when(s < NS - 1)
    def _():
        pltpu.sync_copy(acc_sh.at[pl.ds(cr0, 1568)],
                        acc_out.at[pl.ds(out0 + cr0, 1568)])

    @pl.when(s == NS - 1)
    def _():
        pltpu.sync_copy(acc_sh.at[pl.ds(cr0, 1480)],
                        acc_out.at[pl.ds(out0 + cr0, 1480)])


EPQ = RPT * B
SB = 256
NSB = EPQ // SB
SSB = 4


@functools.partial(
    pl.kernel,
    out_type=(jax.ShapeDtypeStruct((PE,), jnp.float32),
              jax.ShapeDtypeStruct((PE,), jnp.float32)),
    mesh=_mesh,
    compiler_params=pltpu.CompilerParams(use_tc_tiling_on_sc=False,
                                         needs_layout_passes=False),
    scratch_types=[
        pltpu.VMEM((EPQ,), jnp.int32),
        pltpu.VMEM((EPQ,), jnp.int32),
        pltpu.VMEM((SB, D // 2), jnp.int32),
        pltpu.VMEM((SB, D // 2), jnp.int32),
        pltpu.VMEM((SB, D // 2), jnp.int32),
        pltpu.VMEM((SB, D // 2), jnp.int32),
        pltpu.VMEM((L, L + 1), jnp.float32),
        pltpu.VMEM((SSB * SB,), jnp.float32),
        pltpu.SemaphoreType.DMA,
        pltpu.SemaphoreType.DMA,
        pltpu.SemaphoreType.DMA,
        pltpu.SemaphoreType.DMA,
    ],
)
def _score_kernel(z_hbm, pa_hbm, pb_hbm, na_hbm, nb_hbm, pos_out, neg_out,
                  aidx_v, bidx_v, za0_v, za1_v, zb0_v, zb1_v, t2d_v, scr_v,
                  sa0, sa1, sb0, sb1):
    c = lax.axis_index("c")
    s = lax.axis_index("s")
    wid = s * NC + c
    e0 = wid * EPQ
    lanes = lax.iota(jnp.int32, L)

    def _polarity(a_hbm, b_hbm, out_hbm):
        pltpu.sync_copy(a_hbm.at[pl.ds(e0, EPQ)], aidx_v)
        pltpu.sync_copy(b_hbm.at[pl.ds(e0, EPQ)], bidx_v)

        def _issue(sb, za_v, zb_v, sa, sbm):
            base = sb * SB
            pltpu.async_copy(z_hbm.at[aidx_v.at[pl.ds(base, B)]],
                             za_v.at[pl.ds(0, B)], sa)
            pltpu.async_copy(z_hbm.at[aidx_v.at[pl.ds(base + B, B)]],
                             za_v.at[pl.ds(B, B)], sa)
            pltpu.async_copy(z_hbm.at[bidx_v.at[pl.ds(base, B)]],
                             zb_v.at[pl.ds(0, B)], sbm)
            pltpu.async_copy(z_hbm.at[bidx_v.at[pl.ds(base + B, B)]],
                             zb_v.at[pl.ds(B, B)], sbm)

        def _half(sb, j, za_v, zb_v, sa, sbm):
            base = sb * SB
            pltpu.make_async_copy(z_hbm.at[aidx_v.at[pl.ds(base, B)]],
                                  za_v.at[pl.ds(0, B)], sa).wait()
            pltpu.make_async_copy(z_hbm.at[aidx_v.at[pl.ds(base + B, B)]],
                                  za_v.at[pl.ds(B, B)], sa).wait()
            pltpu.make_async_copy(z_hbm.at[bidx_v.at[pl.ds(base, B)]],
                                  zb_v.at[pl.ds(0, B)], sbm).wait()
            pltpu.make_async_copy(z_hbm.at[bidx_v.at[pl.ds(base + B, B)]],
                                  zb_v.at[pl.ds(B, B)], sbm).wait()

            def _sub(i, _):
                for e in range(L):
                    row = jnp.full((L,), i * L + e, jnp.int32)
                    p = _zero16f()
                    q = _zero16f()
                    for k in range(D // (2 * L)):
                        cols = lanes + k * L
                        wa = plsc.load_gather(za_v, [row, cols])
                        wb = plsc.load_gather(zb_v, [row, cols])
                        a0, a1 = plsc.unpack(
                            plsc.bitcast(wa, jnp.bfloat16),
                            format=plsc.PackFormat.INTERLEAVED)
                        b0, b1 = plsc.unpack(
                            plsc.bitcast(wb, jnp.bfloat16),
                            format=plsc.PackFormat.INTERLEAVED)
                        p = p + a0 * b0
                        q = q + a1 * b1
                    t2d_v[e, pl.ds(0, L)] = p + q
                a0 = _zero16f()
                a1 = _zero16f()
                a2 = _zero16f()
                a3 = _zero16f()
                for k in range(L // 4):
                    c0 = jnp.full((L,), 4 * k, jnp.int32)
                    c1 = jnp.full((L,), 4 * k + 1, jnp.int32)
                    c2 = jnp.full((L,), 4 * k + 2, jnp.int32)
                    c3 = jnp.full((L,), 4 * k + 3, jnp.int32)
                    a0 = a0 + plsc.load_gather(t2d_v, [lanes, c0])
                    a1 = a1 + plsc.load_gather(t2d_v, [lanes, c1])
                    a2 = a2 + plsc.load_gather(t2d_v, [lanes, c2])
                    a3 = a3 + plsc.load_gather(t2d_v, [lanes, c3])
                acc = (a0 + a1) + (a2 + a3)
                scr_v[pl.ds((j * (SB // L) + i) * L, L)] = acc
                return 0

            lax.fori_loop(0, SB // L, _sub, 0)

            @pl.when(sb + 2 < NSB)
            def _():
                _issue(sb + 2, za_v, zb_v, sa, sbm)

        def _blk(t, _):
            b0 = t * SSB

            def _pair(p, _):
                j = 2 * p
                _half(b0 + j, j, za0_v, zb0_v, sa0, sb0)
                _half(b0 + j + 1, j + 1, za1_v, zb1_v, sa1, sb1)
                return 0

            lax.fori_loop(0, SSB // 2, _pair, 0)
            pltpu.sync_copy(scr_v,
                            out_hbm.at[pl.ds(e0 + b0 * SB, SSB * SB)])
            return 0

        _issue(0, za0_v, zb0_v, sa0, sb0)
        _issue(1, za1_v, zb1_v, sa1, sb1)
        lax.fori_loop(0, NSB // SSB, _blk, 0)

    _polarity(pa_hbm, pb_hbm, pos_out)
    _polarity(na_hbm, nb_hbm, neg_out)


BN = 5000


def _tc_a_body(dp_ref, x_ref, w1_ref, dinv_ref, g1_ref):
    deg = jnp.sum(dp_ref[...], axis=1) + 1.0
    dinv = lax.rsqrt(deg)
    dinv_ref[...] = dinv[:, None]
    g1_ref[...] = dinv[:, None] * jnp.dot(
        x_ref[...], w1_ref[...], preferred_element_type=jnp.float32,
        precision=lax.Precision.HIGHEST)


_tc_a = pl.pallas_call(
    _tc_a_body,
    grid=(N // BN,),
    in_specs=[
        pl.BlockSpec((BN, NC), lambda i: (i, 0)),
        pl.BlockSpec((BN, D), lambda i: (i, 0)),
        pl.BlockSpec((D, D), lambda i: (0, 0)),
    ],
    out_specs=[
        pl.BlockSpec((BN, 1), lambda i: (i, 0)),
        pl.BlockSpec((BN, D), lambda i: (i, 0)),
    ],
    out_shape=[
        jax.ShapeDtypeStruct((N, 1), jnp.float32),
        jax.ShapeDtypeStruct((N, D), jnp.float32),
    ],
)


def _tc_b_body(acc_ref, g1_ref, dinv_ref, b1_ref, w2_ref, g2_ref):
    dinv = dinv_ref[...]
    h = jax.nn.relu(dinv * (acc_ref[...] + g1_ref[...]) + b1_ref[...])
    g2_ref[...] = dinv * jnp.dot(
        h, w2_ref[...], preferred_element_type=jnp.float32,
        precision=lax.Precision.HIGHEST)


_tc_b = pl.pallas_call(
    _tc_b_body,
    grid=(N // BN,),
    in_specs=[
        pl.BlockSpec((BN, D), lambda i: (i, 0)),
        pl.BlockSpec((BN, D), lambda i: (i, 0)),
        pl.BlockSpec((BN, 1), lambda i: (i, 0)),
        pl.BlockSpec((1, D), lambda i: (0, 0)),
        pl.BlockSpec((D, D), lambda i: (0, 0)),
    ],
    out_specs=pl.BlockSpec((BN, D), lambda i: (i, 0)),
    out_shape=jax.ShapeDtypeStruct((N, D), jnp.float32),
)


def _tc_c_body(acc_ref, g2_ref, dinv_ref, b2_ref, z_ref):
    z_ref[...] = (dinv_ref[...] * (acc_ref[...] + g2_ref[...])
                  + b2_ref[...])


_tc_c = pl.pallas_call(
    _tc_c_body,
    grid=(N // BN,),
    in_specs=[
        pl.BlockSpec((BN, D), lambda i: (i, 0)),
        pl.BlockSpec((BN, D), lambda i: (i, 0)),
        pl.BlockSpec((BN, 1), lambda i: (i, 0)),
        pl.BlockSpec((1, D), lambda i: (0, 0)),
    ],
    out_specs=pl.BlockSpec((BN, D), lambda i: (i, 0)),
    out_shape=jax.ShapeDtypeStruct((N, D), jnp.float32),
)


def _pad_idx(a, fill):
    pad = jnp.full((PE - E,), fill, jnp.int32)
    return jnp.concatenate([a, pad]).reshape(ROWS, B)


def kernel(x, edge_index, pos_edge_index, neg_edge_index, W1, b1, W2, b2):
    dstP = _pad_idx(edge_index[1], -1)
    srcF = _pad_idx(edge_index[0], 0).reshape(PE)
    dstF = dstP.reshape(PE)
    paF = _pad_idx(pos_edge_index[0], 0).reshape(PE)
    pbF = _pad_idx(pos_edge_index[1], 0).reshape(PE)
    naF = _pad_idx(neg_edge_index[0], 0).reshape(PE)
    nbF = _pad_idx(neg_edge_index[1], 0).reshape(PE)

    deg_p = _deg_kernel(dstP)
    dp = jnp.stack([deg_p[:N], deg_p[NP:NP + N]], axis=1)
    dinv2d, g1 = _tc_a(dp, x, W1)
    acc1 = _conv_kernel(g1, srcF, dstF)
    g2 = _tc_b(acc1, g1, dinv2d, b1.reshape(1, D), W2)
    acc2 = _conv_kernel(g2, srcF, dstF)
    z = _tc_c(acc2, g2, dinv2d, b2.reshape(1, D))
    z_bits = lax.bitcast_convert_type(
        z.astype(jnp.bfloat16).reshape(N, D // 2, 2), jnp.int32)
    pos_s, neg_s = _score_kernel(z_bits, paF, pbF, naF, nbF)
    return (pos_s[:E], neg_s[:E])
